# Initial kernel scaffold; baseline (speedup 1.0000x reference)
import jax
import jax.numpy as jnp
from jax.experimental import pallas as pl


def kernel(queries, values, absolute_positions, src_table, tgt_table):
    raise NotImplementedError("write your pallas kernel here")


# speedup vs baseline: 1.3282x; 1.3282x over previous
import functools

import jax
import jax.numpy as jnp
from jax import lax
from jax.experimental import pallas as pl
from jax.experimental.pallas import tpu as pltpu
from jax.experimental.pallas import tpu_sc as plsc

HIDDEN = 128
NLANE = 16


def _pos_encoding(length, hidden_size):
    pos = jnp.arange(length, dtype=jnp.float32)[:, None]
    i = jnp.arange(hidden_size // 2, dtype=jnp.float32)[None, :]
    angle_rates = 1.0 / jnp.power(10000.0, (2.0 * i) / jnp.float32(hidden_size))
    angles = pos * angle_rates
    return jnp.concatenate([jnp.sin(angles), jnp.cos(angles)], axis=-1)


def _sc_gather_call(n_rows, chunk, nchunk, nw):
    rows_per_w = n_rows // nw
    mesh = plsc.VectorSubcoreMesh(core_axis_name="c", subcore_axis_name="s")

    @functools.partial(
        pl.kernel,
        mesh=mesh,
        out_type=[
            jax.ShapeDtypeStruct((nw, nchunk, chunk, HIDDEN), jnp.float32),
            jax.ShapeDtypeStruct((nw, nchunk, chunk, HIDDEN), jnp.float32),
        ],
        scratch_types=[
            pltpu.VMEM((nchunk, chunk), jnp.int32),
            pltpu.VMEM((nchunk, chunk), jnp.int32),
            pltpu.VMEM((chunk, HIDDEN), jnp.float32),
            pltpu.VMEM((chunk, HIDDEN), jnp.float32),
            pltpu.VMEM((chunk, HIDDEN), jnp.float32),
            pltpu.SemaphoreType.DMA,
            pltpu.SemaphoreType.DMA,
        ],
    )
    def k(qidx_hbm, vidx_hbm, tgt_hbm, src_hbm, pos_hbm, qout_hbm, vout_hbm,
          qidx_v, vidx_v, qrows_v, vrows_v, pos_v, qsem, vsem):
        wid = lax.axis_index("s") * 2 + lax.axis_index("c")
        pltpu.sync_copy(qidx_hbm.at[wid], qidx_v)
        pltpu.sync_copy(vidx_hbm.at[wid], vidx_v)
        pltpu.sync_copy(pos_hbm, pos_v)

        def body(c, carry):
            cp_q = pltpu.make_async_copy(tgt_hbm.at[qidx_v.at[c]], qrows_v,
                                         qsem)
            cp_q.start()
            cp_v = pltpu.make_async_copy(src_hbm.at[vidx_v.at[c]], vrows_v,
                                         vsem)
            cp_v.start()
            cp_q.wait()
            pltpu.sync_copy(qrows_v, qout_hbm.at[wid, c])
            cp_v.wait()

            def addrow(r, inner):
                for h in range(HIDDEN // NLANE):
                    sl = pl.ds(h * NLANE, NLANE)
                    vrows_v[r, sl] = vrows_v[r, sl] + pos_v[r, sl]
                return inner

            lax.fori_loop(0, chunk, addrow, 0)
            pltpu.sync_copy(vrows_v, vout_hbm.at[wid, c])
            return carry

        lax.fori_loop(0, nchunk, body, 0)

    return k


def _tc_matmul_call(b, l, nb):
    def body(ap_ref, qr_ref, pos_ref, out_ref):
        ap = ap_ref[...]
        qr = qr_ref[...]
        acc = lax.dot_general(
            ap, qr, (((2,), (1,)), ((0,), (0,))),
            preferred_element_type=jnp.float32)
        out_ref[...] = acc + pos_ref[...]

    return pl.pallas_call(
        body,
        grid=(b // nb,),
        in_specs=[
            pl.BlockSpec((nb, l, l), lambda i: (i, 0, 0)),
            pl.BlockSpec((nb, l, HIDDEN), lambda i: (i, 0, 0)),
            pl.BlockSpec((1, l, HIDDEN), lambda i: (0, 0, 0)),
        ],
        out_specs=pl.BlockSpec((nb, l, HIDDEN), lambda i: (i, 0, 0)),
        out_shape=jax.ShapeDtypeStruct((b, l, HIDDEN), jnp.float32),
    )


def kernel(queries, values, absolute_positions, src_table, tgt_table):
    b, l = queries.shape
    n_rows = b * l
    nw = 32
    rows_per_w = n_rows // nw
    chunk = 100
    nchunk = rows_per_w // chunk

    pos = _pos_encoding(l, HIDDEN)
    pos_tiled = jnp.tile(pos, (chunk // l, 1))

    q_idx = queries.reshape(nw, nchunk, chunk).astype(jnp.int32)
    v_idx = values.reshape(nw, nchunk, chunk).astype(jnp.int32)

    sc = _sc_gather_call(n_rows, chunk, nchunk, nw)
    q_rows, v_emb = sc(q_idx, v_idx, tgt_table, src_table, pos_tiled)

    tc = _tc_matmul_call(b, l, nb=8)
    q_emb = tc(absolute_positions, q_rows.reshape(b, l, HIDDEN),
               pos[None, :, :])
    return q_emb, v_emb.reshape(b, l, HIDDEN)
